# Initial kernel scaffold; baseline (speedup 1.0000x reference)
import jax
import jax.numpy as jnp
from jax.experimental import pallas as pl


def kernel(x, wd, bd, wi_v, bi_v, wh_v, bhn_v, wi_nx, wi_nd, wi_nv, bi_n, wh_n, bhn_n, wi_dx, wi_dv, wi_dn, bi_d, wh_d, bhn_d, wo_v, wo_d, bo):
    raise NotImplementedError("write your pallas kernel here")


# speedup vs baseline: 1.4231x; 1.4231x over previous
import jax
import jax.numpy as jnp
from jax import lax
from jax.experimental import pallas as pl
from jax.experimental.pallas import tpu as pltpu

LANE = 128
GATES = 3
U = 8
FEAT = 44
FEAT_PAD = 128
OUT_COLS = 25
HV, HN = 24, 48


def _mm(a, b):
    return jnp.dot(a, b, preferred_element_type=jnp.float32)


def _shift_gate_cols(w, h, s):
    r = w.shape[0]
    w3 = w.reshape(r, GATES, LANE)
    return jnp.pad(w3[:, :, :h], ((0, 0), (0, 0), (s, LANE - s - h))).reshape(r, GATES * LANE)


def _rnnoise_kernel(
    x_ref,
    wd_ref, bd_ref,
    wi_v_ref, bi_v_ref,
    wi_nx_ref, wi_nd_ref, wi_nv_ref, bi_n_ref,
    wh_vn_ref, bhn_vn_ref,
    wi_dx_ref, bi_d_ref, wi_dvn_ref,
    wh_d_ref, bhn_d_ref,
    wo_v_ref, wo_d_ref, bo_ref,
    out_ref,
    xp_v_s, xp_n_s, xp_d_s, vng_s, deng_s, hvn_s, hd_s,
):
    tq = x_ref.shape[0]
    nb = tq // U

    @pl.when(pl.program_id(0) == 0)
    def _init():
        hvn_s[...] = jnp.zeros_like(hvn_s)
        hd_s[...] = jnp.zeros_like(hd_s)

    x = x_ref[...]
    x_b = x.astype(jnp.bfloat16)

    dense = jnp.tanh(_mm(x, wd_ref[...]) + bd_ref[...])
    dense_b = dense.astype(jnp.bfloat16)

    xp_v_s[...] = _mm(dense_b, wi_v_ref[...]) + bi_v_ref[...]
    xp_n_s[...] = _mm(x_b, wi_nx_ref[...]) + _mm(dense_b, wi_nd_ref[...]) + bi_n_ref[...]
    xp_d_s[...] = _mm(x_b, wi_dx_ref[...]) + bi_d_ref[...]

    wh_vn = wh_vn_ref[...]; bhn_vn = bhn_vn_ref[...]
    wh_d = wh_d_ref[...]; bhn_d = bhn_d_ref[...]
    wi_nv = wi_nv_ref[...]
    wi_dvn = wi_dvn_ref[...]

    vmask1 = lax.broadcasted_iota(jnp.int32, (1, LANE), 1) < HV
    vmask8 = lax.broadcasted_iota(jnp.int32, (U, LANE), 1) < HV

    def _base(b):
        if isinstance(b, int):
            return b * U
        return pl.multiple_of(b * U, U)

    def gru_steps(xp_blk, wh, bhn, h):
        rows = []
        for u in range(U):
            xrow = xp_blk[u:u + 1, :]
            hm = _mm(h.astype(jnp.bfloat16), wh)
            rz = jax.nn.sigmoid(xrow[:, :2 * LANE] + hm[:, :2 * LANE])
            r, z = rz[:, :LANE], rz[:, LANE:]
            n = jnp.tanh(xrow[:, 2 * LANE:] + r * (hm[:, 2 * LANE:] + bhn))
            h = n + z * (h - n)
            rows.append(h)
        return jnp.concatenate(rows, axis=0), h

    def wave(bm, with_noise, bd, prev):
        if bm is not None:
            if bm == "noise_only":
                nbase = _base(nb - 1)
                xp = xp_n_s[pl.ds(nbase, U), :] + _mm(prev.astype(jnp.bfloat16), wi_nv)
            else:
                vbase = _base(bm)
                xp = xp_v_s[pl.ds(vbase, U), :]
                if with_noise:
                    nbase = _base(bm - 1)
                    xp = (xp + xp_n_s[pl.ds(nbase, U), :]
                          + _mm(prev.astype(jnp.bfloat16), wi_nv))
            h0 = hvn_s[...]
        if bd is not None:
            dbase = _base(bd)
            xp_d = (xp_d_s[pl.ds(dbase, U), :]
                    + _mm(vng_s[pl.ds(dbase, U), :].astype(jnp.bfloat16), wi_dvn))
            hd0 = hd_s[...]
        if bm is not None:
            hrows, h1 = gru_steps(xp, wh_vn, bhn_vn, h0)
            act = jnp.where(vmask8, jnp.tanh(hrows), jnp.maximum(hrows, 0.0))
        if bd is not None:
            drows, hd1 = gru_steps(xp_d, wh_d, bhn_d, hd0)
            d_out = jnp.tanh(drows)
        new_prev = prev
        if bm is not None:
            if with_noise:
                vng_s[pl.ds(nbase, U), :] = jnp.where(vmask8, prev, act)
            hvn_s[...] = h1
            new_prev = act
        if bd is not None:
            deng_s[pl.ds(dbase, U), :] = d_out
            hd_s[...] = hd1
        return new_prev

    prev = wave(0, False, None, jnp.zeros((U, LANE), jnp.float32))
    hvn_s[...] = jnp.where(vmask1, hvn_s[...], 0.0)
    prev = wave(1, True, None, prev)

    def body(b, prev):
        return wave(b, True, b - 2, prev)

    prev = lax.fori_loop(2, nb, body, prev, unroll=False)

    h_keep = hvn_s[...]
    prev = wave("noise_only", True, nb - 2, prev)
    hvn_s[...] = jnp.where(vmask1, h_keep, hvn_s[...])
    wave(None, False, nb - 1, prev)

    out_ref[...] = jax.nn.sigmoid(
        _mm(vng_s[...], wo_v_ref[...]) + _mm(deng_s[...], wo_d_ref[...]) + bo_ref[...])


def _forward(x, params, *, tq=512):
    T, F = x.shape
    t_pad = ((T + tq - 1) // tq) * tq
    x = jnp.pad(x.astype(jnp.float32), ((0, t_pad - T), (0, FEAT_PAD - F)))

    def _full(p):
        return pl.BlockSpec(p.shape, lambda i: (0, 0))

    in_specs = ([pl.BlockSpec((tq, FEAT_PAD), lambda i: (i, 0))]
                + [_full(p) for p in params])

    out = pl.pallas_call(
        _rnnoise_kernel,
        out_shape=jax.ShapeDtypeStruct((t_pad, LANE), jnp.float32),
        grid=(t_pad // tq,),
        in_specs=in_specs,
        out_specs=pl.BlockSpec((tq, LANE), lambda i: (i, 0)),
        scratch_shapes=[
            pltpu.VMEM((tq, GATES * LANE), jnp.float32),
            pltpu.VMEM((tq, GATES * LANE), jnp.float32),
            pltpu.VMEM((tq, GATES * LANE), jnp.float32),
            pltpu.VMEM((tq, LANE), jnp.float32),
            pltpu.VMEM((tq, LANE), jnp.float32),
            pltpu.VMEM((1, LANE), jnp.float32),
            pltpu.VMEM((1, LANE), jnp.float32),
        ],
        compiler_params=pltpu.CompilerParams(
            dimension_semantics=("arbitrary",),
            vmem_limit_bytes=64 * 1024 * 1024,
        ),
    )(x, *params)
    return out[:T, 0:1], out[:T, 1:OUT_COLS]


def kernel(x, wd, bd, wi_v, bi_v, wh_v, bhn_v, wi_nx, wi_nd, wi_nv, bi_n,
           wh_n, bhn_n, wi_dx, wi_dv, wi_dn, bi_d, wh_d, bhn_d, wo_v, wo_d, bo):
    wi_nx_s = _shift_gate_cols(wi_nx, HN, HV)
    wi_nd_s = _shift_gate_cols(wi_nd, HN, HV)
    wi_nv_s = _shift_gate_cols(wi_nv, HN, HV)
    bi_n_s = _shift_gate_cols(bi_n, HN, HV)
    wh_n_s = jnp.pad(_shift_gate_cols(wh_n, HN, HV)[:HN], ((HV, LANE - HV - HN), (0, 0)))
    wh_vn = wh_v + wh_n_s
    bhn_vn = bhn_v + jnp.pad(bhn_n[:, :HN], ((0, 0), (HV, LANE - HV - HN)))
    wi_dvn = wi_dv + jnp.pad(wi_dn[:HN], ((HV, LANE - HV - HN), (0, 0)))
    params = (wd, bd, wi_v, bi_v, wi_nx_s, wi_nd_s, wi_nv_s, bi_n_s,
              wh_vn, bhn_vn, wi_dx, bi_d, wi_dvn, wh_d, bhn_d, wo_v, wo_d, bo)
    return _forward(x, params, tq=512)


# speedup vs baseline: 1.5019x; 1.0554x over previous
import jax
import jax.numpy as jnp
from jax import lax
from jax.experimental import pallas as pl
from jax.experimental.pallas import tpu as pltpu

LANE = 128
GATES = 3
U = 8
FEAT = 44
FEAT_PAD = 128
OUT_COLS = 25
HV, HN = 24, 48


def _mm(a, b):
    return jnp.dot(a, b, preferred_element_type=jnp.float32)


def _shift_gate_cols(w, h, s):
    r = w.shape[0]
    w3 = w.reshape(r, GATES, LANE)
    return jnp.pad(w3[:, :, :h], ((0, 0), (0, 0), (s, LANE - s - h))).reshape(r, GATES * LANE)


def _rnnoise_kernel(
    x_ref,
    wd_ref, bd_ref,
    wi_v_ref, bi_v_ref,
    wi_nx_ref, wi_nd_ref, wi_nv_ref, bi_n_ref,
    wh_vn_ref, bhn_vn_ref,
    wi_dx_ref, bi_d_ref, wi_dvn_ref,
    wh_d_ref, bhn_d_ref,
    wo_v_ref, wo_d_ref, bo_ref,
    out_ref,
    xp_v_s, xp_n_s, xp_d_s, vng_s, deng_s, hvn_s, hd_s,
):
    tq = x_ref.shape[0]
    nb = tq // U

    @pl.when(pl.program_id(0) == 0)
    def _init():
        hvn_s[...] = jnp.zeros_like(hvn_s)
        hd_s[...] = jnp.zeros_like(hd_s)

    x = x_ref[...]
    x_b = x.astype(jnp.bfloat16)

    dense = jnp.tanh(_mm(x, wd_ref[...]) + bd_ref[...])
    dense_b = dense.astype(jnp.bfloat16)

    xp_v_s[...] = _mm(dense_b, wi_v_ref[...]) + bi_v_ref[...]
    xp_n_s[...] = _mm(x_b, wi_nx_ref[...]) + _mm(dense_b, wi_nd_ref[...]) + bi_n_ref[...]
    xp_d_s[...] = _mm(x_b, wi_dx_ref[...]) + bi_d_ref[...]

    wh_vn = wh_vn_ref[...]; bhn_vn = bhn_vn_ref[...]
    wh_d = wh_d_ref[...]; bhn_d = bhn_d_ref[...]
    wi_nv = wi_nv_ref[...]
    wi_dvn = wi_dvn_ref[...]

    vmask1 = lax.broadcasted_iota(jnp.int32, (1, LANE), 1) < HV
    vmask8 = lax.broadcasted_iota(jnp.int32, (U, LANE), 1) < HV

    def _base(b):
        if isinstance(b, int):
            return b * U
        return pl.multiple_of(b * U, U)

    def gru_steps(xp_blk, wh, bhn, h):
        rows = []
        for u in range(U):
            xrow = xp_blk[u:u + 1, :]
            hm = _mm(h.astype(jnp.bfloat16), wh)
            th = jnp.tanh(xrow[:, :2 * LANE] + hm[:, :2 * LANE])
            th_r, th_z = th[:, :LANE], th[:, LANE:]
            z = 0.5 * th_z + 0.5
            p = 0.5 * (hm[:, 2 * LANE:] + bhn)
            n = jnp.tanh(xrow[:, 2 * LANE:] + p + th_r * p)
            h = n + z * (h - n)
            rows.append(h)
        return jnp.concatenate(rows, axis=0), h

    def wave(bm, with_noise, bd, prev):
        if bm is not None:
            if bm == "noise_only":
                nbase = _base(nb - 1)
                xp = xp_n_s[pl.ds(nbase, U), :] + _mm(prev.astype(jnp.bfloat16), wi_nv)
            else:
                vbase = _base(bm)
                xp = xp_v_s[pl.ds(vbase, U), :]
                if with_noise:
                    nbase = _base(bm - 1)
                    xp = (xp + xp_n_s[pl.ds(nbase, U), :]
                          + _mm(prev.astype(jnp.bfloat16), wi_nv))
            h0 = hvn_s[...]
        if bd is not None:
            dbase = _base(bd)
            xp_d = (xp_d_s[pl.ds(dbase, U), :]
                    + _mm(vng_s[pl.ds(dbase, U), :].astype(jnp.bfloat16), wi_dvn))
            hd0 = hd_s[...]
        if bm is not None:
            hrows, h1 = gru_steps(xp, wh_vn, bhn_vn, h0)
            act = jnp.where(vmask8, jnp.tanh(hrows), jnp.maximum(hrows, 0.0))
        if bd is not None:
            drows, hd1 = gru_steps(xp_d, wh_d, bhn_d, hd0)
            d_out = jnp.tanh(drows)
        new_prev = prev
        if bm is not None:
            if with_noise:
                vng_s[pl.ds(nbase, U), :] = jnp.where(vmask8, prev, act)
            hvn_s[...] = h1
            new_prev = act
        if bd is not None:
            deng_s[pl.ds(dbase, U), :] = d_out
            hd_s[...] = hd1
        return new_prev

    prev = wave(0, False, None, jnp.zeros((U, LANE), jnp.float32))
    hvn_s[...] = jnp.where(vmask1, hvn_s[...], 0.0)
    prev = wave(1, True, None, prev)

    def body(b, prev):
        return wave(b, True, b - 2, prev)

    prev = lax.fori_loop(2, nb, body, prev, unroll=False)

    h_keep = hvn_s[...]
    prev = wave("noise_only", True, nb - 2, prev)
    hvn_s[...] = jnp.where(vmask1, h_keep, hvn_s[...])
    wave(None, False, nb - 1, prev)

    out_ref[...] = jax.nn.sigmoid(
        _mm(vng_s[...], wo_v_ref[...]) + _mm(deng_s[...], wo_d_ref[...]) + bo_ref[...])


def _forward(x, params, *, tq=512):
    T, F = x.shape
    t_pad = ((T + tq - 1) // tq) * tq
    x = jnp.pad(x.astype(jnp.float32), ((0, t_pad - T), (0, FEAT_PAD - F)))

    def _full(p):
        return pl.BlockSpec(p.shape, lambda i: (0, 0))

    in_specs = ([pl.BlockSpec((tq, FEAT_PAD), lambda i: (i, 0))]
                + [_full(p) for p in params])

    out = pl.pallas_call(
        _rnnoise_kernel,
        out_shape=jax.ShapeDtypeStruct((t_pad, LANE), jnp.float32),
        grid=(t_pad // tq,),
        in_specs=in_specs,
        out_specs=pl.BlockSpec((tq, LANE), lambda i: (i, 0)),
        scratch_shapes=[
            pltpu.VMEM((tq, GATES * LANE), jnp.float32),
            pltpu.VMEM((tq, GATES * LANE), jnp.float32),
            pltpu.VMEM((tq, GATES * LANE), jnp.float32),
            pltpu.VMEM((tq, LANE), jnp.float32),
            pltpu.VMEM((tq, LANE), jnp.float32),
            pltpu.VMEM((1, LANE), jnp.float32),
            pltpu.VMEM((1, LANE), jnp.float32),
        ],
        compiler_params=pltpu.CompilerParams(
            dimension_semantics=("arbitrary",),
            vmem_limit_bytes=64 * 1024 * 1024,
        ),
    )(x, *params)
    return out[:T, 0:1], out[:T, 1:OUT_COLS]


def _halve_rz(w):
    r = w.shape[0]
    w3 = w.reshape(r, GATES, LANE)
    return jnp.concatenate([w3[:, :2] * 0.5, w3[:, 2:]], axis=1).reshape(r, GATES * LANE)


def _repack(wd, bd, wi_v, bi_v, wh_v, bhn_v, wi_nx, wi_nd, wi_nv, bi_n,
            wh_n, bhn_n, wi_dx, wi_dv, wi_dn, bi_d, wh_d, bhn_d, wo_v, wo_d, bo):
    wi_nx_s = _shift_gate_cols(wi_nx, HN, HV)
    wi_nd_s = _shift_gate_cols(wi_nd, HN, HV)
    wi_nv_s = _shift_gate_cols(wi_nv, HN, HV)
    bi_n_s = _shift_gate_cols(bi_n, HN, HV)
    wh_n_s = jnp.pad(_shift_gate_cols(wh_n, HN, HV)[:HN], ((HV, LANE - HV - HN), (0, 0)))
    wh_vn = wh_v + wh_n_s
    bhn_vn = bhn_v + jnp.pad(bhn_n[:, :HN], ((0, 0), (HV, LANE - HV - HN)))
    wi_dvn = wi_dv + jnp.pad(wi_dn[:HN], ((HV, LANE - HV - HN), (0, 0)))
    return (wd, bd, _halve_rz(wi_v), _halve_rz(bi_v), _halve_rz(wi_nx_s),
            _halve_rz(wi_nd_s), _halve_rz(wi_nv_s), _halve_rz(bi_n_s),
            _halve_rz(wh_vn), bhn_vn, _halve_rz(wi_dx), _halve_rz(bi_d),
            _halve_rz(wi_dvn), _halve_rz(wh_d), bhn_d, wo_v, wo_d, bo)


def kernel(x, wd, bd, wi_v, bi_v, wh_v, bhn_v, wi_nx, wi_nd, wi_nv, bi_n,
           wh_n, bhn_n, wi_dx, wi_dv, wi_dn, bi_d, wh_d, bhn_d, wo_v, wo_d, bo):
    params = _repack(wd, bd, wi_v, bi_v, wh_v, bhn_v, wi_nx, wi_nd, wi_nv, bi_n,
                     wh_n, bhn_n, wi_dx, wi_dv, wi_dn, bi_d, wh_d, bhn_d,
                     wo_v, wo_d, bo)
    return _forward(x, params, tq=512)
